# baseline (speedup 1.0000x reference)
import functools

import jax
import jax.numpy as jnp
from jax import lax
from jax.experimental import pallas as pl
from jax.experimental.pallas import tpu as pltpu
from jax.experimental.pallas import tpu_sc as plsc

WEIGHT = 0.15
SIGMA = 1.0
NDCG_K = 10

B = 512
K = 31
N = K + 1
NN = N * N
DE = 129
V = 2048

_HI = jax.lax.Precision.HIGHEST


def _sc_gather(td2, anchor_codes, fi):
    info = plsc.get_sparse_core_info()
    nw = info.num_cores * info.num_subcores
    per_w = B // nw
    mesh = plsc.VectorSubcoreMesh(core_axis_name="c", subcore_axis_name="s")

    @functools.partial(
        pl.kernel,
        mesh=mesh,
        out_type=(
            jax.ShapeDtypeStruct((B * N,), jnp.float32),
            jax.ShapeDtypeStruct((B * V,), jnp.float32),
        ),
        scratch_types=[
            pltpu.VMEM((per_w,), jnp.int32),
            pltpu.VMEM((per_w, V), jnp.float32),
            pltpu.VMEM((per_w * N,), jnp.int32),
            pltpu.VMEM((per_w * N,), jnp.float32),
            pltpu.SemaphoreType.DMA,
            pltpu.SemaphoreType.DMA,
        ],
    )
    def k(td_hbm, ac_hbm, fi_hbm, out_hbm, rows_hbm,
          ac_v, rows_v, fi_v, out_v, sem, sem2):
        wid = lax.axis_index("s") * info.num_cores + lax.axis_index("c")
        base = wid * per_w
        pltpu.sync_copy(ac_hbm.at[pl.ds(base, per_w)], ac_v)
        pltpu.sync_copy(fi_hbm.at[pl.ds(base * N, per_w * N)], fi_v)
        pltpu.async_copy(td_hbm.at[ac_v], rows_v, sem).wait()
        handles = [
            pltpu.async_copy(
                rows_v.at[a], rows_hbm.at[pl.ds((base + a) * V, V)], sem2)
            for a in range(per_w)
        ]
        for h in handles:
            h.wait()
        pltpu.async_copy(rows_hbm.at[fi_v], out_v, sem).wait()
        pltpu.sync_copy(out_v, out_hbm.at[pl.ds(base * N, per_w * N)])

    out, _ = k(td2, anchor_codes, fi)
    return out.reshape(B, N)


_BB = 128
_BK = _BB * K


def _dist(inner):
    z = jnp.maximum(-inner, 1.0 + 1e-7)
    return jnp.log(z + jnp.sqrt((z - 1.0) * (z + 1.0)))


def _tc_dist_body(at_ref, pt_ref, nt_ref, d_ref):
    at = at_ref[...]
    pt = pt_ref[...]
    nt = nt_ref[...]

    rowd = lax.broadcasted_iota(jnp.int32, (DE, 1), 0)
    at0 = jnp.where(rowd == 0, 0.0, at)
    pt0 = jnp.where(rowd == 0, 0.0, pt)
    nt0 = jnp.where(rowd == 0, 0.0, nt)

    asq = jnp.sum(at0 * at0, axis=0, keepdims=True)
    psq = jnp.sum(pt0 * pt0, axis=0, keepdims=True)
    nsq = jnp.sum(nt0 * nt0, axis=0, keepdims=True)
    ta = jnp.sqrt(1.0 + asq)
    tp = jnp.sqrt(1.0 + psq)
    tn = jnp.sqrt(1.0 + nsq)

    ip = jnp.sum(at0 * pt0, axis=0, keepdims=True) - ta * tp
    dp = _dist(ip)

    rb = lax.broadcasted_iota(jnp.int32, (_BB, _BK), 0)
    cj = lax.broadcasted_iota(jnp.int32, (_BB, _BK), 1)
    e31 = ((cj >= rb * K) & (cj < rb * K + K)).astype(jnp.float32)
    at_rep = jax.lax.dot(at0, e31, precision=_HI)
    ta_rep = jax.lax.dot(ta, e31, precision=_HI)
    inn = jnp.sum(at_rep * nt0, axis=0, keepdims=True) - ta_rep * tn
    dn = _dist(inn)

    rr = lax.broadcasted_iota(jnp.int32, (_BB, _BB), 0)
    cc = lax.broadcasted_iota(jnp.int32, (_BB, _BB), 1)
    eye_dp = jnp.where(rr == cc, 1.0, 0.0) * dp
    ones1 = jnp.ones((_BB, 1), jnp.float32)
    d1 = jax.lax.dot(eye_dp, ones1, precision=_HI)
    jj = lax.broadcasted_iota(jnp.int32, (_BK, K), 0)
    qk = lax.broadcasted_iota(jnp.int32, (_BK, K), 1)
    q = (lax.rem(jj, K) == qk).astype(jnp.float32)
    d2 = jax.lax.dot(e31 * dn, q, precision=_HI)
    d_ref[...] = jnp.concatenate([d1, d2], axis=1)


def _tc_dist(a_t, p_t, n_t):
    grid = B // _BB
    return pl.pallas_call(
        _tc_dist_body,
        grid=(grid,),
        in_specs=[
            pl.BlockSpec((DE, _BB), lambda i: (0, i)),
            pl.BlockSpec((DE, _BB), lambda i: (0, i)),
            pl.BlockSpec((DE, _BK), lambda i: (0, i)),
        ],
        out_specs=pl.BlockSpec((_BB, N), lambda i: (i, 0)),
        out_shape=jax.ShapeDtypeStruct((B, N), jnp.float32),
    )(a_t, p_t, n_t)


def _tc_pair_body(d_ref, td_ref, out_ref):
    dist = d_ref[...]
    td = td_ref[...]

    maxtd = jnp.max(td, axis=1, keepdims=True)
    rel = (maxtd - td + 1e-6) / (maxtd + 1e-6)

    row = lax.broadcasted_iota(jnp.int32, (N, NN), 0)
    colp = lax.broadcasted_iota(jnp.int32, (N, NN), 1)
    ei = (lax.shift_right_logical(colp, 5) == row).astype(jnp.float32)
    ej = ((colp & (N - 1)) == row).astype(jnp.float32)

    def bcast(x, e):
        return jax.lax.dot(x, e, precision=_HI)

    def pair_reduce(x, e):
        return lax.dot_general(x, e, (((1,), (1,)), ((), ())), precision=_HI)

    iip = lax.shift_right_logical(
        lax.broadcasted_iota(jnp.int32, (1, NN), 1), 5)
    jjp = lax.broadcasted_iota(jnp.int32, (1, NN), 1) & (N - 1)
    tie = (jjp < iip).astype(jnp.float32)
    upper = iip < jjp

    di = bcast(dist, ei)
    dj = bcast(dist, ej)
    lt = jnp.where(dj < di, 1.0, 0.0) + jnp.where(dj == di, tie, 0.0)
    ranks = pair_reduce(lt, ei)
    g = jnp.where(ranks < NDCG_K, 1.0 / jnp.log2(ranks + 2.0), 0.0)

    ri = bcast(rel, ei)
    rj = bcast(rel, ej)
    rlt = jnp.where(rj > ri, 1.0, 0.0) + jnp.where(rj == ri, tie, 0.0)
    rranks = pair_reduce(rlt, ei)
    rdisc = jnp.where(rranks < NDCG_K, 1.0 / jnp.log2(rranks + 2.0), 0.0)
    ideal = jnp.sum(rel * rdisc, axis=1, keepdims=True)

    gi = bcast(g, ei)
    gj = bcast(g, ej)
    delta = jnp.abs((ri - rj) * (gj - gi)) / jnp.maximum(ideal, 1e-30)
    delta = jnp.where(ideal > 0.0, delta, 0.0)
    prob = 1.0 / (1.0 + jnp.exp(SIGMA * (dj - di)))
    lam = jnp.where(
        ri > rj, delta * (1.0 - prob),
        jnp.where(rj > ri, -delta * prob, 0.0),
    )
    part = jnp.sum(jnp.where(upper, lam * (di - dj), 0.0)) * (WEIGHT / B)
    out_ref[...] = jnp.full((1, 1), part, jnp.float32)


def _tc_pair(all_d, all_td):
    return pl.pallas_call(
        _tc_pair_body,
        out_shape=jax.ShapeDtypeStruct((1, 1), jnp.float32),
    )(all_d, all_td)


def kernel(anchor_emb, positive_emb, negative_embs, tree_distances,
           anchor_codes, positive_codes, negative_codes,
           batch_size, k_negatives):
    all_codes = jnp.concatenate(
        [positive_codes[:, None], negative_codes], axis=1)
    fi = (jnp.arange(B, dtype=jnp.int32)[:, None] * V
          + all_codes.astype(jnp.int32)).reshape(-1)

    all_td = _sc_gather(tree_distances, anchor_codes.astype(jnp.int32), fi)

    all_d = _tc_dist(anchor_emb.T, positive_emb.T, negative_embs.T)

    out = _tc_pair(all_d, all_td)
    return out[0, 0]


# speedup vs baseline: 1.1175x; 1.1175x over previous
import functools

import jax
import jax.numpy as jnp
from jax import lax
from jax.experimental import pallas as pl
from jax.experimental.pallas import tpu as pltpu
from jax.experimental.pallas import tpu_sc as plsc

WEIGHT = 0.15
SIGMA = 1.0
NDCG_K = 10

B = 512
K = 31
N = K + 1
NN = N * N
DE = 129
V = 2048

_HI = jax.lax.Precision.HIGHEST


def _sc_gather(td2, anchor_codes, fi):
    info = plsc.get_sparse_core_info()
    nw = info.num_cores * info.num_subcores
    per_w = B // nw
    mesh = plsc.VectorSubcoreMesh(core_axis_name="c", subcore_axis_name="s")

    @functools.partial(
        pl.kernel,
        mesh=mesh,
        out_type=(
            jax.ShapeDtypeStruct((B * N,), jnp.float32),
            jax.ShapeDtypeStruct((B * V,), jnp.float32),
        ),
        scratch_types=[
            pltpu.VMEM((per_w,), jnp.int32),
            pltpu.VMEM((per_w, V), jnp.float32),
            pltpu.VMEM((per_w * N,), jnp.int32),
            pltpu.VMEM((per_w * N,), jnp.float32),
            pltpu.SemaphoreType.DMA,
            pltpu.SemaphoreType.DMA,
        ],
    )
    def k(td_hbm, ac_hbm, fi_hbm, out_hbm, rows_hbm,
          ac_v, rows_v, fi_v, out_v, sem, sem2):
        wid = lax.axis_index("s") * info.num_cores + lax.axis_index("c")
        base = wid * per_w
        pltpu.sync_copy(ac_hbm.at[pl.ds(base, per_w)], ac_v)
        pltpu.sync_copy(fi_hbm.at[pl.ds(base * N, per_w * N)], fi_v)
        pltpu.async_copy(td_hbm.at[ac_v], rows_v, sem).wait()
        handles = [
            pltpu.async_copy(
                rows_v.at[a], rows_hbm.at[pl.ds((base + a) * V, V)], sem2)
            for a in range(per_w)
        ]
        for h in handles:
            h.wait()
        pltpu.async_copy(rows_hbm.at[fi_v], out_v, sem).wait()
        pltpu.sync_copy(out_v, out_hbm.at[pl.ds(base * N, per_w * N)])

    out, _ = k(td2, anchor_codes, fi)
    return out.reshape(B, N)


_BB = 128
_BK = _BB * K


def _dist(inner):
    z = jnp.maximum(-inner, 1.0 + 1e-7)
    return jnp.log(z + jnp.sqrt((z - 1.0) * (z + 1.0)))


def _tc_dist_body(at_ref, pt_ref, nt_ref, dp_ref, dn_ref):
    at = at_ref[...]
    pt = pt_ref[...]
    nt = nt_ref[...]

    rowd = lax.broadcasted_iota(jnp.int32, (DE, 1), 0)
    at0 = jnp.where(rowd == 0, 0.0, at)
    pt0 = jnp.where(rowd == 0, 0.0, pt)
    nt0 = jnp.where(rowd == 0, 0.0, nt)

    asq = jnp.sum(at0 * at0, axis=0, keepdims=True)
    psq = jnp.sum(pt0 * pt0, axis=0, keepdims=True)
    nsq = jnp.sum(nt0 * nt0, axis=0, keepdims=True)
    ta = jnp.sqrt(1.0 + asq)
    tp = jnp.sqrt(1.0 + psq)
    tn = jnp.sqrt(1.0 + nsq)

    ip = jnp.sum(at0 * pt0, axis=0, keepdims=True) - ta * tp
    dp_ref[...] = _dist(ip)

    rb = lax.broadcasted_iota(jnp.int32, (_BB, _BK), 0)
    cj = lax.broadcasted_iota(jnp.int32, (_BB, _BK), 1)
    e31 = ((cj >= rb * K) & (cj < rb * K + K)).astype(jnp.float32)
    at_rep = jax.lax.dot(at0, e31, precision=_HI)
    ta_rep = jax.lax.dot(ta, e31, precision=_HI)
    inn = jnp.sum(at_rep * nt0, axis=0, keepdims=True) - ta_rep * tn
    dn_ref[...] = _dist(inn)


def _tc_dist(a_t, p_t, n_t):
    grid = B // _BB
    return pl.pallas_call(
        _tc_dist_body,
        grid=(grid,),
        in_specs=[
            pl.BlockSpec((DE, _BB), lambda i: (0, i)),
            pl.BlockSpec((DE, _BB), lambda i: (0, i)),
            pl.BlockSpec((DE, _BK), lambda i: (0, i)),
        ],
        out_specs=[
            pl.BlockSpec((1, _BB), lambda i: (0, i)),
            pl.BlockSpec((1, _BK), lambda i: (0, i)),
        ],
        out_shape=[
            jax.ShapeDtypeStruct((1, B), jnp.float32),
            jax.ShapeDtypeStruct((1, B * K), jnp.float32),
        ],
    )(a_t, p_t, n_t)


def _tc_pair_body(d_ref, td_ref, out_ref):
    dist = d_ref[...]
    td = td_ref[...]

    maxtd = jnp.max(td, axis=1, keepdims=True)
    rel = (maxtd - td + 1e-6) / (maxtd + 1e-6)

    row = lax.broadcasted_iota(jnp.int32, (N, NN), 0)
    colp = lax.broadcasted_iota(jnp.int32, (N, NN), 1)
    ei = (lax.shift_right_logical(colp, 5) == row).astype(jnp.float32)
    ej = ((colp & (N - 1)) == row).astype(jnp.float32)

    def bcast(x, e):
        return jax.lax.dot(x, e, precision=_HI)

    def pair_reduce(x, e):
        return lax.dot_general(x, e, (((1,), (1,)), ((), ())), precision=_HI)

    iip = lax.shift_right_logical(
        lax.broadcasted_iota(jnp.int32, (1, NN), 1), 5)
    jjp = lax.broadcasted_iota(jnp.int32, (1, NN), 1) & (N - 1)
    tie = (jjp < iip).astype(jnp.float32)
    upper = iip < jjp

    di = bcast(dist, ei)
    dj = bcast(dist, ej)
    lt = jnp.where(dj < di, 1.0, 0.0) + jnp.where(dj == di, tie, 0.0)
    ranks = pair_reduce(lt, ei)
    g = jnp.where(ranks < NDCG_K, 1.0 / jnp.log2(ranks + 2.0), 0.0)

    ri = bcast(rel, ei)
    rj = bcast(rel, ej)
    rlt = jnp.where(rj > ri, 1.0, 0.0) + jnp.where(rj == ri, tie, 0.0)
    rranks = pair_reduce(rlt, ei)
    rdisc = jnp.where(rranks < NDCG_K, 1.0 / jnp.log2(rranks + 2.0), 0.0)
    ideal = jnp.sum(rel * rdisc, axis=1, keepdims=True)

    gi = bcast(g, ei)
    gj = bcast(g, ej)
    delta = jnp.abs((ri - rj) * (gj - gi)) / jnp.maximum(ideal, 1e-30)
    delta = jnp.where(ideal > 0.0, delta, 0.0)
    prob = 1.0 / (1.0 + jnp.exp(SIGMA * (dj - di)))
    lam = jnp.where(
        ri > rj, delta * (1.0 - prob),
        jnp.where(rj > ri, -delta * prob, 0.0),
    )
    part = jnp.sum(jnp.where(upper, lam * (di - dj), 0.0)) * (WEIGHT / B)
    out_ref[...] = jnp.full((1, 1), part, jnp.float32)


def _tc_pair(all_d, all_td):
    return pl.pallas_call(
        _tc_pair_body,
        out_shape=jax.ShapeDtypeStruct((1, 1), jnp.float32),
    )(all_d, all_td)


def kernel(anchor_emb, positive_emb, negative_embs, tree_distances,
           anchor_codes, positive_codes, negative_codes,
           batch_size, k_negatives):
    all_codes = jnp.concatenate(
        [positive_codes[:, None], negative_codes], axis=1)
    fi = (jnp.arange(B, dtype=jnp.int32)[:, None] * V
          + all_codes.astype(jnp.int32)).reshape(-1)

    all_td = _sc_gather(tree_distances, anchor_codes.astype(jnp.int32), fi)

    d_pos, d_neg = _tc_dist(anchor_emb.T, positive_emb.T, negative_embs.T)
    all_d = jnp.concatenate(
        [d_pos.reshape(B, 1), d_neg.reshape(B, K)], axis=1)

    out = _tc_pair(all_d, all_td)
    return out[0, 0]


# speedup vs baseline: 1.2780x; 1.1436x over previous
import functools

import jax
import jax.numpy as jnp
from jax import lax
from jax.experimental import pallas as pl
from jax.experimental.pallas import tpu as pltpu
from jax.experimental.pallas import tpu_sc as plsc

WEIGHT = 0.15
SIGMA = 1.0
NDCG_K = 10

B = 512
K = 31
N = K + 1
NN = N * N
DE = 129
V = 2048

_HI = jax.lax.Precision.HIGHEST


def _sc_gather(td2, anchor_codes, fi):
    info = plsc.get_sparse_core_info()
    nw = info.num_cores * info.num_subcores
    per_w = B // nw
    mesh = plsc.VectorSubcoreMesh(core_axis_name="c", subcore_axis_name="s")

    @functools.partial(
        pl.kernel,
        mesh=mesh,
        out_type=(
            jax.ShapeDtypeStruct((B * N,), jnp.float32),
            jax.ShapeDtypeStruct((B * V,), jnp.float32),
        ),
        scratch_types=[
            pltpu.VMEM((per_w,), jnp.int32),
            pltpu.VMEM((per_w, V), jnp.float32),
            pltpu.VMEM((per_w * N,), jnp.int32),
            pltpu.VMEM((per_w * N,), jnp.float32),
            pltpu.SemaphoreType.DMA,
            pltpu.SemaphoreType.DMA,
        ],
    )
    def k(td_hbm, ac_hbm, fi_hbm, out_hbm, rows_hbm,
          ac_v, rows_v, fi_v, out_v, sem, sem2):
        wid = lax.axis_index("s") * info.num_cores + lax.axis_index("c")
        base = wid * per_w
        pltpu.sync_copy(ac_hbm.at[pl.ds(base, per_w)], ac_v)
        pltpu.sync_copy(fi_hbm.at[pl.ds(base * N, per_w * N)], fi_v)
        pltpu.async_copy(td_hbm.at[ac_v], rows_v, sem).wait()
        handles = [
            pltpu.async_copy(
                rows_v.at[a], rows_hbm.at[pl.ds((base + a) * V, V)], sem2)
            for a in range(per_w)
        ]
        for h in handles:
            h.wait()
        pltpu.async_copy(rows_hbm.at[fi_v], out_v, sem).wait()
        pltpu.sync_copy(out_v, out_hbm.at[pl.ds(base * N, per_w * N)])

    out, _ = k(td2, anchor_codes, fi)
    return out.reshape(B, N)


_BB = 128
_BK = _BB * K


def _dist(inner):
    z = jnp.maximum(-inner, 1.0 + 1e-7)
    return jnp.log(z + jnp.sqrt((z - 1.0) * (z + 1.0)))


def _tc_dist_body(at_ref, pt_ref, nt_ref, dp_ref, dn_ref):
    at = at_ref[...]
    pt = pt_ref[...]
    nt = nt_ref[...]

    rowd = lax.broadcasted_iota(jnp.int32, (DE, 1), 0)
    at0 = jnp.where(rowd == 0, 0.0, at)
    pt0 = jnp.where(rowd == 0, 0.0, pt)
    nt0 = jnp.where(rowd == 0, 0.0, nt)

    asq = jnp.sum(at0 * at0, axis=0, keepdims=True)
    psq = jnp.sum(pt0 * pt0, axis=0, keepdims=True)
    nsq = jnp.sum(nt0 * nt0, axis=0, keepdims=True)
    ta = jnp.sqrt(1.0 + asq)
    tp = jnp.sqrt(1.0 + psq)
    tn = jnp.sqrt(1.0 + nsq)

    ip = jnp.sum(at0 * pt0, axis=0, keepdims=True) - ta * tp
    dp_ref[...] = _dist(ip)

    rb = lax.broadcasted_iota(jnp.int32, (_BB, _BK), 0)
    cj = lax.broadcasted_iota(jnp.int32, (_BB, _BK), 1)
    e31 = ((cj >= rb * K) & (cj < rb * K + K)).astype(jnp.float32)
    at_rep = jax.lax.dot(at0, e31, precision=_HI)
    ta_rep = jax.lax.dot(ta, e31, precision=_HI)
    inn = jnp.sum(at_rep * nt0, axis=0, keepdims=True) - ta_rep * tn
    dn_ref[...] = _dist(inn)


def _tc_dist(a_t, p_t, n_t):
    grid = B // _BB
    return pl.pallas_call(
        _tc_dist_body,
        grid=(grid,),
        in_specs=[
            pl.BlockSpec((DE, _BB), lambda i: (0, i)),
            pl.BlockSpec((DE, _BB), lambda i: (0, i)),
            pl.BlockSpec((DE, _BK), lambda i: (0, i)),
        ],
        out_specs=[
            pl.BlockSpec((1, _BB), lambda i: (0, i)),
            pl.BlockSpec((1, _BK), lambda i: (0, i)),
        ],
        out_shape=[
            jax.ShapeDtypeStruct((1, B), jnp.float32),
            jax.ShapeDtypeStruct((1, B * K), jnp.float32),
        ],
    )(a_t, p_t, n_t)


def _tc_pair_body(d_ref, td_ref, out_ref):
    dist = d_ref[...]
    td = td_ref[...]

    maxtd = jnp.max(td, axis=1, keepdims=True)
    rel = (maxtd - td + 1e-6) / (maxtd + 1e-6)

    row = lax.broadcasted_iota(jnp.int32, (N, NN), 0)
    colp = lax.broadcasted_iota(jnp.int32, (N, NN), 1)
    ei = (lax.shift_right_logical(colp, 5) == row).astype(jnp.float32)
    ej = ((colp & (N - 1)) == row).astype(jnp.float32)
    eij = ei - ej

    iip = lax.shift_right_logical(
        lax.broadcasted_iota(jnp.int32, (1, NN), 1), 5)
    jjp = lax.broadcasted_iota(jnp.int32, (1, NN), 1) & (N - 1)
    tie = (jjp < iip).astype(jnp.float32)

    dr = jnp.concatenate([dist, rel], axis=0)
    dif = jax.lax.dot(dr, eij, precision=_HI)
    dd = dif[:B]
    rdd = dif[B:]

    lt = jnp.where(dd > 0.0, 1.0, 0.0) + jnp.where(dd == 0.0, tie, 0.0)
    rlt = jnp.where(rdd < 0.0, 1.0, 0.0) + jnp.where(rdd == 0.0, tie, 0.0)
    both = jnp.concatenate([lt, rlt], axis=0)
    ranks2 = lax.dot_general(both, ei, (((1,), (1,)), ((), ())))
    ranks = ranks2[:B]
    rranks = ranks2[B:]
    g = jnp.where(ranks < NDCG_K, 1.0 / jnp.log2(ranks + 2.0), 0.0)
    rdisc = jnp.where(rranks < NDCG_K, 1.0 / jnp.log2(rranks + 2.0), 0.0)
    ideal = jnp.sum(rel * rdisc, axis=1, keepdims=True)

    gdd = jax.lax.dot(g, eij, precision=_HI)
    delta = jnp.abs(rdd * gdd) / jnp.maximum(ideal, 1e-30)
    delta = jnp.where(ideal > 0.0, delta, 0.0)
    u = jnp.sign(rdd) * (SIGMA * dd)
    part = jnp.sum(delta * u / (1.0 + jnp.exp(u))) * (0.5 * WEIGHT / B)
    out_ref[...] = jnp.full((1, 1), part, jnp.float32)


def _tc_pair(all_d, all_td):
    return pl.pallas_call(
        _tc_pair_body,
        out_shape=jax.ShapeDtypeStruct((1, 1), jnp.float32),
    )(all_d, all_td)


def kernel(anchor_emb, positive_emb, negative_embs, tree_distances,
           anchor_codes, positive_codes, negative_codes,
           batch_size, k_negatives):
    all_codes = jnp.concatenate(
        [positive_codes[:, None], negative_codes], axis=1)
    fi = (jnp.arange(B, dtype=jnp.int32)[:, None] * V
          + all_codes.astype(jnp.int32)).reshape(-1)

    all_td = _sc_gather(tree_distances, anchor_codes.astype(jnp.int32), fi)

    d_pos, d_neg = _tc_dist(anchor_emb.T, positive_emb.T, negative_embs.T)
    all_d = jnp.concatenate(
        [d_pos.reshape(B, 1), d_neg.reshape(B, K)], axis=1)

    out = _tc_pair(all_d, all_td)
    return out[0, 0]


# speedup vs baseline: 1.3700x; 1.0720x over previous
import functools

import jax
import jax.numpy as jnp
from jax import lax
from jax.experimental import pallas as pl
from jax.experimental.pallas import tpu as pltpu
from jax.experimental.pallas import tpu_sc as plsc

WEIGHT = 0.15
SIGMA = 1.0
NDCG_K = 10

B = 512
K = 31
N = K + 1
NN = N * N
DE = 129
V = 2048

_HI = jax.lax.Precision.HIGHEST


def _sc_gather(td2, anchor_codes, fi):
    info = plsc.get_sparse_core_info()
    nw = info.num_cores * info.num_subcores
    per_w = B // nw
    mesh = plsc.VectorSubcoreMesh(core_axis_name="c", subcore_axis_name="s")

    @functools.partial(
        pl.kernel,
        mesh=mesh,
        out_type=(
            jax.ShapeDtypeStruct((B * N,), jnp.float32),
            jax.ShapeDtypeStruct((B * V,), jnp.float32),
        ),
        scratch_types=[
            pltpu.VMEM((per_w,), jnp.int32),
            pltpu.VMEM((per_w, V), jnp.float32),
            pltpu.VMEM((per_w * N,), jnp.int32),
            pltpu.VMEM((per_w * N,), jnp.float32),
            pltpu.SemaphoreType.DMA,
            pltpu.SemaphoreType.DMA,
        ],
    )
    def k(td_hbm, ac_hbm, fi_hbm, out_hbm, rows_hbm,
          ac_v, rows_v, fi_v, out_v, sem, sem2):
        wid = lax.axis_index("s") * info.num_cores + lax.axis_index("c")
        base = wid * per_w
        pltpu.sync_copy(ac_hbm.at[pl.ds(base, per_w)], ac_v)
        pltpu.sync_copy(fi_hbm.at[pl.ds(base * N, per_w * N)], fi_v)
        pltpu.async_copy(td_hbm.at[ac_v], rows_v, sem).wait()
        handles = [
            pltpu.async_copy(
                rows_v.at[a], rows_hbm.at[pl.ds((base + a) * V, V)], sem2)
            for a in range(per_w)
        ]
        for h in handles:
            h.wait()
        pltpu.async_copy(rows_hbm.at[fi_v], out_v, sem).wait()
        pltpu.sync_copy(out_v, out_hbm.at[pl.ds(base * N, per_w * N)])

    out, _ = k(td2, anchor_codes, fi)
    return out.reshape(B, N)


_BB = 128
_BK = _BB * K


def _dot3(x, e):
    hi = x.astype(jnp.bfloat16).astype(jnp.float32)
    r1 = x - hi
    mid = r1.astype(jnp.bfloat16).astype(jnp.float32)
    lo = r1 - mid
    return (jax.lax.dot(hi, e) + jax.lax.dot(mid, e)
            + jax.lax.dot(lo, e))


def _dist(inner):
    z = jnp.maximum(-inner, 1.0 + 1e-7)
    return jnp.log(z + jnp.sqrt((z - 1.0) * (z + 1.0)))


def _tc_dist_body(at_ref, pt_ref, nt_ref, d_ref):
    at = at_ref[...]
    pt = pt_ref[...]
    nt = nt_ref[...]

    rowd = lax.broadcasted_iota(jnp.int32, (DE, 1), 0)
    at0 = jnp.where(rowd == 0, 0.0, at)
    pt0 = jnp.where(rowd == 0, 0.0, pt)
    nt0 = jnp.where(rowd == 0, 0.0, nt)

    asq = jnp.sum(at0 * at0, axis=0, keepdims=True)
    psq = jnp.sum(pt0 * pt0, axis=0, keepdims=True)
    nsq = jnp.sum(nt0 * nt0, axis=0, keepdims=True)
    ta = jnp.sqrt(1.0 + asq)
    tp = jnp.sqrt(1.0 + psq)
    tn = jnp.sqrt(1.0 + nsq)

    ip = jnp.sum(at0 * pt0, axis=0, keepdims=True) - ta * tp
    dp = _dist(ip)

    rb = lax.broadcasted_iota(jnp.int32, (_BB, _BK), 0)
    cj = lax.broadcasted_iota(jnp.int32, (_BB, _BK), 1)
    e31 = ((cj >= rb * K) & (cj < rb * K + K)).astype(jnp.float32)
    at_rep = _dot3(at0, e31)
    ta_rep = _dot3(ta, e31)
    inn = jnp.sum(at_rep * nt0, axis=0, keepdims=True) - ta_rep * tn
    dn = _dist(inn)

    rr = lax.broadcasted_iota(jnp.int32, (_BB, _BB), 0)
    cc = lax.broadcasted_iota(jnp.int32, (_BB, _BB), 1)
    eye = jnp.where(rr == cc, 1.0, 0.0)
    ones1 = jnp.ones((_BB, 1), jnp.float32)
    d1 = _dot3(eye * dp, ones1)
    jj = lax.broadcasted_iota(jnp.int32, (_BK, K), 0)
    qk = lax.broadcasted_iota(jnp.int32, (_BK, K), 1)
    q = (lax.rem(jj, K) == qk).astype(jnp.float32)
    d2 = _dot3(e31 * dn, q)
    d_ref[...] = jnp.concatenate([d1, d2], axis=1)


def _tc_dist(a_t, p_t, n_t):
    grid = B // _BB
    return pl.pallas_call(
        _tc_dist_body,
        grid=(grid,),
        in_specs=[
            pl.BlockSpec((DE, _BB), lambda i: (0, i)),
            pl.BlockSpec((DE, _BB), lambda i: (0, i)),
            pl.BlockSpec((DE, _BK), lambda i: (0, i)),
        ],
        out_specs=pl.BlockSpec((_BB, N), lambda i: (i, 0)),
        out_shape=jax.ShapeDtypeStruct((B, N), jnp.float32),
    )(a_t, p_t, n_t)


def _tc_pair_body(d_ref, td_ref, out_ref):
    dist = d_ref[...]
    td = td_ref[...]

    maxtd = jnp.max(td, axis=1, keepdims=True)
    rel = (maxtd - td + 1e-6) / (maxtd + 1e-6)

    row = lax.broadcasted_iota(jnp.int32, (N, NN), 0)
    colp = lax.broadcasted_iota(jnp.int32, (N, NN), 1)
    ei = (lax.shift_right_logical(colp, 5) == row).astype(jnp.float32)
    ej = ((colp & (N - 1)) == row).astype(jnp.float32)
    eij = ei - ej

    iip = lax.shift_right_logical(
        lax.broadcasted_iota(jnp.int32, (1, NN), 1), 5)
    jjp = lax.broadcasted_iota(jnp.int32, (1, NN), 1) & (N - 1)
    tie = (jjp < iip).astype(jnp.float32)

    dr = jnp.concatenate([dist, rel], axis=0)
    dif = _dot3(dr, eij)
    dd = dif[:B]
    rdd = dif[B:]

    lt = jnp.where(dd > 0.0, 1.0, 0.0) + jnp.where(dd == 0.0, tie, 0.0)
    rlt = jnp.where(rdd < 0.0, 1.0, 0.0) + jnp.where(rdd == 0.0, tie, 0.0)
    both = jnp.concatenate([lt, rlt], axis=0)
    ranks2 = lax.dot_general(both, ei, (((1,), (1,)), ((), ())))
    ranks = ranks2[:B]
    rranks = ranks2[B:]
    g = jnp.where(ranks < NDCG_K, 1.0 / jnp.log2(ranks + 2.0), 0.0)
    rdisc = jnp.where(rranks < NDCG_K, 1.0 / jnp.log2(rranks + 2.0), 0.0)
    ideal = jnp.sum(rel * rdisc, axis=1, keepdims=True)

    gdd = _dot3(g, eij)
    delta = jnp.abs(rdd * gdd) / jnp.maximum(ideal, 1e-30)
    delta = jnp.where(ideal > 0.0, delta, 0.0)
    u = jnp.sign(rdd) * (SIGMA * dd)
    part = jnp.sum(delta * u / (1.0 + jnp.exp(u))) * (0.5 * WEIGHT / B)
    out_ref[...] = jnp.full((1, 1), part, jnp.float32)


def _tc_pair(all_d, all_td):
    return pl.pallas_call(
        _tc_pair_body,
        out_shape=jax.ShapeDtypeStruct((1, 1), jnp.float32),
    )(all_d, all_td)


def kernel(anchor_emb, positive_emb, negative_embs, tree_distances,
           anchor_codes, positive_codes, negative_codes,
           batch_size, k_negatives):
    all_codes = jnp.concatenate(
        [positive_codes[:, None], negative_codes], axis=1)
    fi = (jnp.arange(B, dtype=jnp.int32)[:, None] * V
          + all_codes.astype(jnp.int32)).reshape(-1)

    all_td = _sc_gather(tree_distances, anchor_codes.astype(jnp.int32), fi)

    all_d = _tc_dist(anchor_emb.T, positive_emb.T, negative_embs.T)

    out = _tc_pair(all_d, all_td)
    return out[0, 0]


# speedup vs baseline: 1.3720x; 1.0014x over previous
import functools

import jax
import jax.numpy as jnp
from jax import lax
from jax.experimental import pallas as pl
from jax.experimental.pallas import tpu as pltpu
from jax.experimental.pallas import tpu_sc as plsc

WEIGHT = 0.15
SIGMA = 1.0
NDCG_K = 10

B = 512
K = 31
N = K + 1
NN = N * N
DE = 129
V = 2048


def _sc_gather(td2, anchor_codes, fi):
    info = plsc.get_sparse_core_info()
    nw = info.num_cores * info.num_subcores
    per_w = B // nw
    mesh = plsc.VectorSubcoreMesh(core_axis_name="c", subcore_axis_name="s")

    @functools.partial(
        pl.kernel,
        mesh=mesh,
        out_type=(
            jax.ShapeDtypeStruct((B * N,), jnp.float32),
            jax.ShapeDtypeStruct((B * V,), jnp.float32),
        ),
        scratch_types=[
            pltpu.VMEM((per_w,), jnp.int32),
            pltpu.VMEM((per_w, V), jnp.float32),
            pltpu.VMEM((per_w * N,), jnp.int32),
            pltpu.VMEM((per_w * N,), jnp.float32),
            pltpu.SemaphoreType.DMA,
            pltpu.SemaphoreType.DMA,
        ],
    )
    def k(td_hbm, ac_hbm, fi_hbm, out_hbm, rows_hbm,
          ac_v, rows_v, fi_v, out_v, sem, sem2):
        wid = lax.axis_index("s") * info.num_cores + lax.axis_index("c")
        base = wid * per_w
        pltpu.sync_copy(ac_hbm.at[pl.ds(base, per_w)], ac_v)
        pltpu.sync_copy(fi_hbm.at[pl.ds(base * N, per_w * N)], fi_v)
        pltpu.async_copy(td_hbm.at[ac_v], rows_v, sem).wait()
        handles = [
            pltpu.async_copy(
                rows_v.at[a], rows_hbm.at[pl.ds((base + a) * V, V)], sem2)
            for a in range(per_w)
        ]
        for h in handles:
            h.wait()
        pltpu.async_copy(rows_hbm.at[fi_v], out_v, sem).wait()
        pltpu.sync_copy(out_v, out_hbm.at[pl.ds(base * N, per_w * N)])

    out, _ = k(td2, anchor_codes, fi)
    return out.reshape(B, N)


_BB = 128
_BK = _BB * K


def _dot3(x, e):
    hi = x.astype(jnp.bfloat16).astype(jnp.float32)
    r1 = x - hi
    mid = r1.astype(jnp.bfloat16).astype(jnp.float32)
    lo = r1 - mid
    return (jax.lax.dot(hi, e) + jax.lax.dot(mid, e)
            + jax.lax.dot(lo, e))


def _dist(inner):
    z = jnp.maximum(-inner, 1.0 + 1e-7)
    return jnp.log(z + jnp.sqrt((z - 1.0) * (z + 1.0)))


def _tc_dist_body(at_ref, pt_ref, nt_ref, d_ref):
    at = at_ref[...]
    pt = pt_ref[...]
    nt = nt_ref[...]

    rowd = lax.broadcasted_iota(jnp.int32, (DE, 1), 0)
    at0 = jnp.where(rowd == 0, 0.0, at)
    pt0 = jnp.where(rowd == 0, 0.0, pt)
    nt0 = jnp.where(rowd == 0, 0.0, nt)

    asq = jnp.sum(at0 * at0, axis=0, keepdims=True)
    psq = jnp.sum(pt0 * pt0, axis=0, keepdims=True)
    nsq = jnp.sum(nt0 * nt0, axis=0, keepdims=True)
    ta = jnp.sqrt(1.0 + asq)
    tp = jnp.sqrt(1.0 + psq)
    tn = jnp.sqrt(1.0 + nsq)

    ip = jnp.sum(at0 * pt0, axis=0, keepdims=True) - ta * tp
    dp = _dist(ip)

    rb = lax.broadcasted_iota(jnp.int32, (_BB, _BK), 0)
    cj = lax.broadcasted_iota(jnp.int32, (_BB, _BK), 1)
    e31 = ((cj >= rb * K) & (cj < rb * K + K)).astype(jnp.float32)
    at_rep = _dot3(at0, e31)
    ta_rep = _dot3(ta, e31)
    inn = jnp.sum(at_rep * nt0, axis=0, keepdims=True) - ta_rep * tn
    dn = _dist(inn)

    rr = lax.broadcasted_iota(jnp.int32, (_BB, _BB), 0)
    cc = lax.broadcasted_iota(jnp.int32, (_BB, _BB), 1)
    eye = jnp.where(rr == cc, 1.0, 0.0)
    ones1 = jnp.ones((_BB, 1), jnp.float32)
    d1 = _dot3(eye * dp, ones1)
    jj = lax.broadcasted_iota(jnp.int32, (_BK, K), 0)
    qk = lax.broadcasted_iota(jnp.int32, (_BK, K), 1)
    q = (lax.rem(jj, K) == qk).astype(jnp.float32)
    d2 = _dot3(e31 * dn, q)
    d_ref[...] = jnp.concatenate([d1, d2], axis=1)


def _tc_dist(a_t, p_t, n_t):
    grid = B // _BB
    return pl.pallas_call(
        _tc_dist_body,
        grid=(grid,),
        in_specs=[
            pl.BlockSpec((DE, _BB), lambda i: (0, i)),
            pl.BlockSpec((DE, _BB), lambda i: (0, i)),
            pl.BlockSpec((DE, _BK), lambda i: (0, i)),
        ],
        out_specs=pl.BlockSpec((_BB, N), lambda i: (i, 0)),
        out_shape=jax.ShapeDtypeStruct((B, N), jnp.float32),
    )(a_t, p_t, n_t)


def _tc_pair_body(d_ref, td_ref, out_ref):
    dist = d_ref[...]
    td = td_ref[...]

    maxtd = jnp.max(td, axis=1, keepdims=True)
    rel = (maxtd - td + 1e-6) / (maxtd + 1e-6)

    row = lax.broadcasted_iota(jnp.int32, (N, NN), 0)
    colp = lax.broadcasted_iota(jnp.int32, (N, NN), 1)
    ei = (lax.shift_right_logical(colp, 5) == row).astype(jnp.float32)
    ej = ((colp & (N - 1)) == row).astype(jnp.float32)
    eij = ei - ej

    iip = lax.shift_right_logical(
        lax.broadcasted_iota(jnp.int32, (1, NN), 1), 5)
    jjp = lax.broadcasted_iota(jnp.int32, (1, NN), 1) & (N - 1)
    tie = (jjp < iip).astype(jnp.float32)

    dr = jnp.concatenate([dist, rel], axis=0)
    dif = _dot3(dr, eij)
    dd = dif[:B]
    rdd = dif[B:]

    lt = jnp.where(dd > 0.0, 1.0, 0.0) + jnp.where(dd == 0.0, tie, 0.0)
    rlt = jnp.where(rdd < 0.0, 1.0, 0.0) + jnp.where(rdd == 0.0, tie, 0.0)
    both = jnp.concatenate([lt, rlt], axis=0)
    ranks2 = lax.dot_general(both, ei, (((1,), (1,)), ((), ())))
    ranks = ranks2[:B]
    rranks = ranks2[B:]
    g = jnp.where(ranks < NDCG_K, 1.0 / jnp.log2(ranks + 2.0), 0.0)
    rdisc = jnp.where(rranks < NDCG_K, 1.0 / jnp.log2(rranks + 2.0), 0.0)
    ideal = jnp.sum(rel * rdisc, axis=1, keepdims=True)

    gdd = _dot3(g, eij)
    delta = jnp.abs(rdd * gdd) / jnp.maximum(ideal, 1e-30)
    delta = jnp.where(ideal > 0.0, delta, 0.0)
    u = jnp.sign(rdd) * (SIGMA * dd)
    part = jnp.sum(delta * u / (1.0 + jnp.exp(u))) * (0.5 * WEIGHT / B)
    out_ref[...] = jnp.full((1, 1), part, jnp.float32)


def _tc_pair(all_d, all_td):
    return pl.pallas_call(
        _tc_pair_body,
        out_shape=jax.ShapeDtypeStruct((1, 1), jnp.float32),
    )(all_d, all_td)


def kernel(anchor_emb, positive_emb, negative_embs, tree_distances,
           anchor_codes, positive_codes, negative_codes,
           batch_size, k_negatives):
    all_codes = jnp.concatenate(
        [positive_codes[:, None], negative_codes], axis=1)
    fi = (jnp.arange(B, dtype=jnp.int32)[:, None] * V
          + all_codes.astype(jnp.int32)).reshape(-1)

    all_td = _sc_gather(tree_distances, anchor_codes.astype(jnp.int32), fi)

    all_d = _tc_dist(anchor_emb.T, positive_emb.T, negative_embs.T)

    out = _tc_pair(all_d, all_td)
    return out[0, 0]
